# Initial kernel scaffold; baseline (speedup 1.0000x reference)
import jax
import jax.numpy as jnp
from jax.experimental import pallas as pl


def kernel(x, node_attrs, edge_index, atomic_numbers, covalent_radii, a, b):
    raise NotImplementedError("write your pallas kernel here")


# speedup vs baseline: 686.9815x; 686.9815x over previous
import functools

import jax
import jax.numpy as jnp
from jax import lax
from jax.experimental import pallas as pl
from jax.experimental.pallas import tpu as pltpu
from jax.experimental.pallas import tpu_sc as plsc

_NC = 2
_NS = 16
_NW = _NC * _NS
_L = 16


def _node_radius_kernel(attrs_ref, az_ref, cr_ref, out_ref):
    attrs = attrs_ref[...]
    k = lax.broadcasted_iota(jnp.int32, attrs.shape, 1)
    m = jnp.max(attrs, axis=1, keepdims=True)
    first = jnp.min(jnp.where(attrs == m, k, attrs.shape[1]), axis=1,
                    keepdims=True)
    z = jnp.sum(jnp.where(k == first, az_ref[...], 0), axis=1, keepdims=True)
    r = jnp.sum(jnp.where(k == z, cr_ref[...], 0.0), axis=1, keepdims=True)
    out_ref[...] = 0.25 * r


def _soft_kernel(x_ref, r0_ref, ab_ref, y_ref):
    x = x_ref[...]
    u = x / r0_ref[...]
    a = ab_ref[0]
    b = ab_ref[1]
    p = jnp.exp(b * jnp.log(jnp.maximum(u, 1e-30)))
    p = jnp.where(u > 0.0, p, 0.0)
    y_ref[...] = x + 0.5 * jnp.tanh(-u - a * p) + 0.5


def _make_gather_r0(E, N):
    RB = 40
    C = RB * 128
    rows = E // 128
    total_chunks = rows // RB
    mesh = plsc.VectorSubcoreMesh(core_axis_name="c", subcore_axis_name="s")

    @functools.partial(
        pl.kernel,
        mesh=mesh,
        compiler_params=pltpu.CompilerParams(needs_layout_passes=False),
        out_type=jax.ShapeDtypeStruct((rows, 128), jnp.float32),
        scratch_types=[
            pltpu.VMEM((N,), jnp.float32),
            pltpu.VMEM((2, C), jnp.int32),
            pltpu.VMEM((RB, 128), jnp.float32),
        ],
    )
    def gather_r0(rnode_hbm, eidx_hbm, out_hbm, table_v, eb_v, r0_v):
        wid = lax.axis_index("s") * _NC + lax.axis_index("c")
        pltpu.sync_copy(rnode_hbm, table_v)
        my_chunks = (total_chunks - wid + _NW - 1) // _NW

        def chunk_body(ci, carry):
            chunk = wid + ci * _NW
            base = pl.multiple_of(chunk * C, 128)
            pltpu.sync_copy(eidx_hbm.at[:, pl.ds(base, C)], eb_v)

            def row_body(row, c2):
                for j in range(128 // _L):
                    o = row * 128 + j * _L
                    s16 = eb_v[0, pl.ds(o, _L)]
                    r16 = eb_v[1, pl.ds(o, _L)]
                    g = (plsc.load_gather(table_v, [s16]) +
                         plsc.load_gather(table_v, [r16]))
                    r0_v[row, pl.ds(j * _L, _L)] = g
                return c2

            lax.fori_loop(0, RB, row_body, 0)
            rb = pl.multiple_of(chunk * RB, 8)
            pltpu.sync_copy(r0_v, out_hbm.at[pl.ds(rb, RB), :])
            return carry

        lax.fori_loop(0, my_chunks, chunk_body, 0)

    return gather_r0


def kernel(x, node_attrs, edge_index, atomic_numbers, covalent_radii, a, b):
    N, K = node_attrs.shape
    E = x.shape[0]

    az = atomic_numbers.reshape(1, K).astype(jnp.int32)
    crp = jnp.zeros((1, K), jnp.float32)
    crp = crp.at[0, : covalent_radii.shape[0]].set(covalent_radii)

    R = 2000
    rnode4 = pl.pallas_call(
        _node_radius_kernel,
        grid=(N // R,),
        in_specs=[
            pl.BlockSpec((R, K), lambda i: (i, 0)),
            pl.BlockSpec((1, K), lambda i: (0, 0)),
            pl.BlockSpec((1, K), lambda i: (0, 0)),
        ],
        out_specs=pl.BlockSpec((R, 1), lambda i: (i, 0)),
        out_shape=jax.ShapeDtypeStruct((N, 1), jnp.float32),
    )(node_attrs, az, crp)

    r0 = _make_gather_r0(E, N)(rnode4.reshape(N), edge_index)

    W = 128
    rows = E // W
    RB = 2000
    ab = jnp.stack([a.astype(jnp.float32), b.astype(jnp.float32)])
    y = pl.pallas_call(
        _soft_kernel,
        grid=(rows // RB,),
        in_specs=[
            pl.BlockSpec((RB, W), lambda i: (i, 0)),
            pl.BlockSpec((RB, W), lambda i: (i, 0)),
            pl.BlockSpec(memory_space=pltpu.SMEM),
        ],
        out_specs=pl.BlockSpec((RB, W), lambda i: (i, 0)),
        out_shape=jax.ShapeDtypeStruct((rows, W), jnp.float32),
    )(x.reshape(rows, W), r0, ab)
    return y.reshape(E, 1)


# speedup vs baseline: 1285.8819x; 1.8718x over previous
import functools

import jax
import jax.numpy as jnp
from jax import lax
from jax.experimental import pallas as pl
from jax.experimental.pallas import tpu as pltpu
from jax.experimental.pallas import tpu_sc as plsc

_NC = 2
_NS = 16
_NW = _NC * _NS
_L = 16


def _node_radius_kernel(attrs_ref, azf_ref, cr_ref, out_ref, cr4_ref):
    @pl.when(pl.program_id(0) == 0)
    def _():
        z128 = lax.broadcasted_iota(jnp.int32, (128, 128), 1).astype(
            jnp.float32)
        mz = (azf_ref[...] == z128).astype(jnp.float32)
        cr4_ref[...] = 0.25 * jnp.dot(mz, cr_ref[...],
                                      preferred_element_type=jnp.float32)

    attrs = attrs_ref[...]
    kf = lax.broadcasted_iota(jnp.int32, attrs.shape, 1).astype(jnp.float32)
    m = jnp.max(attrs, axis=1, keepdims=True)
    w = jnp.where(attrs == m, 128.0 - kf, 0.0)
    idxf = 128.0 - jnp.max(w, axis=1, keepdims=True)
    oh = (kf == idxf).astype(jnp.float32)
    cr4 = cr4_ref[...]
    dn = (((0,), (1,)), ((), ()))
    rows = [
        lax.dot_general(cr4, oh[128 * r:128 * (r + 1), :], dn,
                        preferred_element_type=jnp.float32)
        for r in range(attrs.shape[0] // 128)
    ]
    out_ref[...] = jnp.concatenate(rows, axis=0)


def _soft_kernel(x_ref, r0_ref, ab_ref, y_ref):
    x = x_ref[...]
    u = x / r0_ref[...]
    a = ab_ref[0]
    b = ab_ref[1]
    p = jnp.exp(b * jnp.log(jnp.maximum(u, 1e-30)))
    p = jnp.where(u > 0.0, p, 0.0)
    y_ref[...] = x + 0.5 * jnp.tanh(-u - a * p) + 0.5


def _make_gather_r0(E, TBL):
    RB = 40
    C = RB * 128
    rows = E // 128
    total_chunks = rows // RB
    mesh = plsc.VectorSubcoreMesh(core_axis_name="c", subcore_axis_name="s")

    @functools.partial(
        pl.kernel,
        mesh=mesh,
        compiler_params=pltpu.CompilerParams(needs_layout_passes=False),
        out_type=jax.ShapeDtypeStruct((rows, 128), jnp.float32),
        scratch_types=[
            pltpu.VMEM((TBL,), jnp.float32),
            pltpu.VMEM((2, C), jnp.int32),
            pltpu.VMEM((2, C), jnp.int32),
            pltpu.VMEM((RB, 128), jnp.float32),
            pltpu.VMEM((RB, 128), jnp.float32),
            pltpu.SemaphoreType.DMA,
            pltpu.SemaphoreType.DMA,
            pltpu.SemaphoreType.DMA,
            pltpu.SemaphoreType.DMA,
            pltpu.SemaphoreType.DMA,
        ],
    )
    def gather_r0(rnode_hbm, eidx_hbm, out_hbm, table_v,
                  eb0, eb1, r00, r01, tsem, is0, is1, os0, os1):
        wid = lax.axis_index("s") * _NC + lax.axis_index("c")
        my_chunks = (total_chunks - wid + _NW - 1) // _NW
        ebs, r0s, isems, osems = (eb0, eb1), (r00, r01), (is0, is1), (os0, os1)

        def in_src(ci):
            chunk = wid + ci * _NW
            base = pl.multiple_of(chunk * C, 128)
            return eidx_hbm.at[:, pl.ds(base, C)]

        def out_dst(ci):
            chunk = wid + ci * _NW
            rb = pl.multiple_of(chunk * RB, 8)
            return out_hbm.at[pl.ds(rb, RB), :]

        @pl.when(my_chunks > 0)
        def _():
            pltpu.async_copy(in_src(0), eb0, is0)

        @pl.when(my_chunks > 1)
        def _():
            pltpu.async_copy(in_src(1), eb1, is1)

        pltpu.async_copy(rnode_hbm.at[pl.ds(0, TBL)], table_v, tsem).wait()

        def pair_body(p, carry):
            for b in range(2):
                ci = p * 2 + b
                eb, r0v, isem, osem = ebs[b], r0s[b], isems[b], osems[b]

                @pl.when(ci < my_chunks)
                def _():
                    pltpu.make_async_copy(in_src(ci), eb, isem).wait()

                    @pl.when(ci >= 2)
                    def _():
                        pltpu.make_async_copy(r0v, out_dst(ci - 2), osem).wait()

                    @plsc.parallel_loop(0, RB, unroll=4)
                    def row_body(row):
                        for j in range(128 // _L):
                            o = row * 128 + j * _L
                            s16 = eb[0, pl.ds(o, _L)]
                            r16 = eb[1, pl.ds(o, _L)]
                            g = (plsc.load_gather(table_v, [s16]) +
                                 plsc.load_gather(table_v, [r16]))
                            r0v[row, pl.ds(j * _L, _L)] = g

                    pltpu.async_copy(r0v, out_dst(ci), osem)

                    @pl.when(ci + 2 < my_chunks)
                    def _():
                        pltpu.async_copy(in_src(ci + 2), eb, isem)

            return carry

        lax.fori_loop(0, (my_chunks + 1) // 2, pair_body, 0)

        for b in range(2):
            @pl.when(my_chunks > b)
            def _():
                pltpu.make_async_copy(r0s[b], out_dst(b), osems[b]).wait()

    return gather_r0


def kernel(x, node_attrs, edge_index, atomic_numbers, covalent_radii, a, b):
    N, K = node_attrs.shape
    E = x.shape[0]
    BN = 1024
    n_blocks = (N + BN - 1) // BN
    N_pad = n_blocks * BN
    TBL = ((N + 127) // 128 + 7) // 8 * 8 * 128

    azf = atomic_numbers.astype(jnp.float32).reshape(K, 1)
    crp = jnp.zeros((K, 1), jnp.float32)
    crp = crp.at[: covalent_radii.shape[0], 0].set(covalent_radii)

    rnode4 = pl.pallas_call(
        _node_radius_kernel,
        grid=(n_blocks,),
        in_specs=[
            pl.BlockSpec((BN, K), lambda i: (i, 0)),
            pl.BlockSpec((K, 1), lambda i: (0, 0)),
            pl.BlockSpec((K, 1), lambda i: (0, 0)),
        ],
        out_specs=pl.BlockSpec((8, 128), lambda i: (i, 0)),
        out_shape=jax.ShapeDtypeStruct((N_pad // 128, 128), jnp.float32),
        scratch_shapes=[pltpu.VMEM((128, 1), jnp.float32)],
    )(node_attrs, azf, crp)

    r0 = _make_gather_r0(E, TBL)(rnode4.reshape(N_pad), edge_index)

    W = 128
    rows = E // W
    RB = 2000
    ab = jnp.stack([a.astype(jnp.float32), b.astype(jnp.float32)])
    y = pl.pallas_call(
        _soft_kernel,
        grid=(rows // RB,),
        in_specs=[
            pl.BlockSpec((RB, W), lambda i: (i, 0)),
            pl.BlockSpec((RB, W), lambda i: (i, 0)),
            pl.BlockSpec(memory_space=pltpu.SMEM),
        ],
        out_specs=pl.BlockSpec((RB, W), lambda i: (i, 0)),
        out_shape=jax.ShapeDtypeStruct((rows, W), jnp.float32),
    )(x.reshape(rows, W), r0, ab)
    return y.reshape(E, 1)


# speedup vs baseline: 1750.4324x; 1.3613x over previous
import functools

import jax
import jax.numpy as jnp
from jax import lax
from jax.experimental import pallas as pl
from jax.experimental.pallas import tpu as pltpu
from jax.experimental.pallas import tpu_sc as plsc

_NC = 2
_NS = 16
_NW = _NC * _NS
_L = 16


def _node_radius_kernel(attrs_ref, azf_ref, cr_ref, out_ref, cr4_ref):
    @pl.when(pl.program_id(0) == 0)
    def _():
        z128 = lax.broadcasted_iota(jnp.int32, (128, 128), 1).astype(
            jnp.float32)
        mz = (azf_ref[...] == z128).astype(jnp.float32)
        cr4_ref[...] = 0.25 * jnp.dot(mz, cr_ref[...],
                                      preferred_element_type=jnp.float32)

    attrs = attrs_ref[...]
    kf = lax.broadcasted_iota(jnp.int32, attrs.shape, 1).astype(jnp.float32)
    m = jnp.max(attrs, axis=1, keepdims=True)
    w = jnp.where(attrs == m, 128.0 - kf, 0.0)
    idxf = 128.0 - jnp.max(w, axis=1, keepdims=True)
    oh = (kf == idxf).astype(jnp.float32)
    cr4 = cr4_ref[...]
    dn = (((0,), (1,)), ((), ()))
    rows = [
        lax.dot_general(cr4, oh[128 * r:128 * (r + 1), :], dn,
                        preferred_element_type=jnp.float32)
        for r in range(attrs.shape[0] // 128)
    ]
    out_ref[...] = jnp.concatenate(rows, axis=0)


def _soft_kernel(x_ref, r0_ref, ab_ref, y_ref):
    x = x_ref[...]
    u = x / r0_ref[...]
    a = ab_ref[0]
    b = ab_ref[1]
    p = jnp.exp(b * jnp.log(jnp.maximum(u, 1e-30)))
    p = jnp.where(u > 0.0, p, 0.0)
    y_ref[...] = x + 0.5 * jnp.tanh(-u - a * p) + 0.5


def _make_gather_r0(E, TBL):
    RB = 40
    C = RB * 128
    rows = E // 128
    total_chunks = rows // RB
    mesh = plsc.VectorSubcoreMesh(core_axis_name="c", subcore_axis_name="s")

    @functools.partial(
        pl.kernel,
        mesh=mesh,
        compiler_params=pltpu.CompilerParams(needs_layout_passes=False),
        out_type=jax.ShapeDtypeStruct((rows, 128), jnp.float32),
        scratch_types=[
            pltpu.VMEM((TBL,), jnp.float32),
            pltpu.VMEM((2, C), jnp.int32),
            pltpu.VMEM((2, C), jnp.int32),
            pltpu.VMEM((RB, 128), jnp.float32),
            pltpu.VMEM((RB, 128), jnp.float32),
            pltpu.SemaphoreType.DMA,
            pltpu.SemaphoreType.DMA,
            pltpu.SemaphoreType.DMA,
            pltpu.SemaphoreType.DMA,
            pltpu.SemaphoreType.DMA,
        ],
    )
    def gather_r0(rnode_hbm, eidx_hbm, out_hbm, table_v,
                  eb0, eb1, r00, r01, tsem, is0, is1, os0, os1):
        wid = lax.axis_index("s") * _NC + lax.axis_index("c")
        my_chunks = (total_chunks - wid + _NW - 1) // _NW
        ebs, r0s, isems, osems = (eb0, eb1), (r00, r01), (is0, is1), (os0, os1)

        def in_src(ci):
            chunk = wid + ci * _NW
            base = pl.multiple_of(chunk * C, 128)
            return eidx_hbm.at[:, pl.ds(base, C)]

        def out_dst(ci):
            chunk = wid + ci * _NW
            rb = pl.multiple_of(chunk * RB, 8)
            return out_hbm.at[pl.ds(rb, RB), :]

        @pl.when(my_chunks > 0)
        def _():
            pltpu.async_copy(in_src(0), eb0, is0)

        @pl.when(my_chunks > 1)
        def _():
            pltpu.async_copy(in_src(1), eb1, is1)

        pltpu.async_copy(rnode_hbm.at[pl.ds(0, TBL)], table_v, tsem).wait()

        def pair_body(p, carry):
            for b in range(2):
                ci = p * 2 + b
                eb, r0v, isem, osem = ebs[b], r0s[b], isems[b], osems[b]

                @pl.when(ci < my_chunks)
                def _():
                    pltpu.make_async_copy(in_src(ci), eb, isem).wait()

                    @pl.when(ci >= 2)
                    def _():
                        pltpu.make_async_copy(r0v, out_dst(ci - 2), osem).wait()

                    @plsc.parallel_loop(0, RB, unroll=4)
                    def row_body(row):
                        for j in range(128 // _L):
                            o = row * 128 + j * _L
                            s16 = eb[0, pl.ds(o, _L)]
                            r16 = eb[1, pl.ds(o, _L)]
                            g = (plsc.load_gather(table_v, [s16]) +
                                 plsc.load_gather(table_v, [r16]))
                            r0v[row, pl.ds(j * _L, _L)] = g

                    pltpu.async_copy(r0v, out_dst(ci), osem)

                    @pl.when(ci + 2 < my_chunks)
                    def _():
                        pltpu.async_copy(in_src(ci + 2), eb, isem)

            return carry

        lax.fori_loop(0, (my_chunks + 1) // 2, pair_body, 0)

        for b in range(2):
            @pl.when(my_chunks > b)
            def _():
                pltpu.make_async_copy(r0s[b], out_dst(b), osems[b]).wait()

    return gather_r0


def kernel(x, node_attrs, edge_index, atomic_numbers, covalent_radii, a, b):
    N, K = node_attrs.shape
    E = x.shape[0]
    BN = 10240
    n_blocks = (N + BN - 1) // BN
    N_pad = n_blocks * BN
    TBL = ((N + 127) // 128 + 7) // 8 * 8 * 128

    azf = atomic_numbers.astype(jnp.float32).reshape(K, 1)
    crp = jnp.zeros((K, 1), jnp.float32)
    crp = crp.at[: covalent_radii.shape[0], 0].set(covalent_radii)

    rnode4 = pl.pallas_call(
        _node_radius_kernel,
        grid=(n_blocks,),
        in_specs=[
            pl.BlockSpec((BN, K), lambda i: (i, 0)),
            pl.BlockSpec((K, 1), lambda i: (0, 0)),
            pl.BlockSpec((K, 1), lambda i: (0, 0)),
        ],
        out_specs=pl.BlockSpec((BN // 128, 128), lambda i: (i, 0)),
        out_shape=jax.ShapeDtypeStruct((N_pad // 128, 128), jnp.float32),
        scratch_shapes=[pltpu.VMEM((128, 1), jnp.float32)],
    )(node_attrs, azf, crp)

    r0 = _make_gather_r0(E, TBL)(rnode4.reshape(N_pad), edge_index)

    W = 128
    rows = E // W
    RB = 2000
    ab = jnp.stack([a.astype(jnp.float32), b.astype(jnp.float32)])
    y = pl.pallas_call(
        _soft_kernel,
        grid=(rows // RB,),
        in_specs=[
            pl.BlockSpec((RB, W), lambda i: (i, 0)),
            pl.BlockSpec((RB, W), lambda i: (i, 0)),
            pl.BlockSpec(memory_space=pltpu.SMEM),
        ],
        out_specs=pl.BlockSpec((RB, W), lambda i: (i, 0)),
        out_shape=jax.ShapeDtypeStruct((rows, W), jnp.float32),
    )(x.reshape(rows, W), r0, ab)
    return y.reshape(E, 1)


# speedup vs baseline: 1841.9204x; 1.0523x over previous
import functools

import jax
import jax.numpy as jnp
from jax import lax
from jax.experimental import pallas as pl
from jax.experimental.pallas import tpu as pltpu
from jax.experimental.pallas import tpu_sc as plsc

_NC = 2
_NS = 16
_NW = _NC * _NS
_L = 16


def _node_radius_kernel(attrs_ref, azf_ref, cr_ref, out_ref, cr4_ref):
    @pl.when(pl.program_id(0) == 0)
    def _():
        z128 = lax.broadcasted_iota(jnp.int32, (128, 128), 1).astype(
            jnp.float32)
        mz = (azf_ref[...] == z128).astype(jnp.float32)
        cr4_ref[...] = 0.25 * jnp.dot(mz, cr_ref[...],
                                      preferred_element_type=jnp.float32)

    attrs = attrs_ref[...]
    kf = lax.broadcasted_iota(jnp.int32, attrs.shape, 1).astype(jnp.float32)
    m = jnp.max(attrs, axis=1, keepdims=True)
    w = jnp.where(attrs == m, 128.0 - kf, 0.0)
    idxf = 128.0 - jnp.max(w, axis=1, keepdims=True)
    oh = (kf == idxf).astype(jnp.float32)
    cr4 = cr4_ref[...]
    dn = (((0,), (1,)), ((), ()))
    rows = [
        lax.dot_general(cr4, oh[128 * r:128 * (r + 1), :], dn,
                        preferred_element_type=jnp.float32)
        for r in range(attrs.shape[0] // 128)
    ]
    out_ref[...] = jnp.concatenate(rows, axis=0)


def _soft_kernel(x_ref, r0_ref, ab_ref, y_ref):
    x = x_ref[...]
    u = x / r0_ref[...]
    a = ab_ref[0]
    b = ab_ref[1]
    p = jnp.exp(b * jnp.log(jnp.maximum(u, 1e-30)))
    p = jnp.where(u > 0.0, p, 0.0)
    y_ref[...] = x + 0.5 * jnp.tanh(-u - a * p) + 0.5


def _make_gather_r0(E, TBL):
    RB = 40
    C = RB * 128
    rows = E // 128
    total_chunks = rows // RB
    mesh = plsc.VectorSubcoreMesh(core_axis_name="c", subcore_axis_name="s")

    @functools.partial(
        pl.kernel,
        mesh=mesh,
        compiler_params=pltpu.CompilerParams(needs_layout_passes=False),
        out_type=jax.ShapeDtypeStruct((rows, 128), jnp.float32),
        scratch_types=[
            pltpu.VMEM((TBL,), jnp.float32),
            pltpu.VMEM((2, C), jnp.int32),
            pltpu.VMEM((2, C), jnp.int32),
            pltpu.VMEM((RB, 128), jnp.float32),
            pltpu.VMEM((RB, 128), jnp.float32),
            pltpu.SemaphoreType.DMA,
            pltpu.SemaphoreType.DMA,
            pltpu.SemaphoreType.DMA,
            pltpu.SemaphoreType.DMA,
            pltpu.SemaphoreType.DMA,
        ],
    )
    def gather_r0(rnode_hbm, eidx_hbm, out_hbm, table_v,
                  eb0, eb1, r00, r01, tsem, is0, is1, os0, os1):
        wid = lax.axis_index("s") * _NC + lax.axis_index("c")
        my_chunks = (total_chunks - wid + _NW - 1) // _NW
        ebs, r0s, isems, osems = (eb0, eb1), (r00, r01), (is0, is1), (os0, os1)

        def in_src(ci):
            chunk = wid + ci * _NW
            base = pl.multiple_of(chunk * C, 128)
            return eidx_hbm.at[:, pl.ds(base, C)]

        def out_dst(ci):
            chunk = wid + ci * _NW
            rb = pl.multiple_of(chunk * RB, 8)
            return out_hbm.at[pl.ds(rb, RB), :]

        @pl.when(my_chunks > 0)
        def _():
            pltpu.async_copy(in_src(0), eb0, is0)

        @pl.when(my_chunks > 1)
        def _():
            pltpu.async_copy(in_src(1), eb1, is1)

        pltpu.async_copy(rnode_hbm.at[pl.ds(0, TBL)], table_v, tsem).wait()

        def pair_body(p, carry):
            for b in range(2):
                ci = p * 2 + b
                eb, r0v, isem, osem = ebs[b], r0s[b], isems[b], osems[b]

                @pl.when(ci < my_chunks)
                def _():
                    pltpu.make_async_copy(in_src(ci), eb, isem).wait()

                    @pl.when(ci >= 2)
                    def _():
                        pltpu.make_async_copy(r0v, out_dst(ci - 2), osem).wait()

                    @plsc.parallel_loop(0, RB, unroll=8)
                    def row_body(row):
                        for j in range(128 // _L):
                            o = row * 128 + j * _L
                            s16 = eb[0, pl.ds(o, _L)]
                            r16 = eb[1, pl.ds(o, _L)]
                            g = (plsc.load_gather(table_v, [s16]) +
                                 plsc.load_gather(table_v, [r16]))
                            r0v[row, pl.ds(j * _L, _L)] = g

                    pltpu.async_copy(r0v, out_dst(ci), osem)

                    @pl.when(ci + 2 < my_chunks)
                    def _():
                        pltpu.async_copy(in_src(ci + 2), eb, isem)

            return carry

        lax.fori_loop(0, (my_chunks + 1) // 2, pair_body, 0)

        for b in range(2):
            @pl.when(my_chunks > b)
            def _():
                pltpu.make_async_copy(r0s[b], out_dst(b), osems[b]).wait()

    return gather_r0


def kernel(x, node_attrs, edge_index, atomic_numbers, covalent_radii, a, b):
    N, K = node_attrs.shape
    E = x.shape[0]
    BN = 10240
    n_blocks = (N + BN - 1) // BN
    N_pad = n_blocks * BN
    TBL = ((N + 127) // 128 + 7) // 8 * 8 * 128

    azf = atomic_numbers.astype(jnp.float32).reshape(K, 1)
    crp = jnp.zeros((K, 1), jnp.float32)
    crp = crp.at[: covalent_radii.shape[0], 0].set(covalent_radii)

    rnode4 = pl.pallas_call(
        _node_radius_kernel,
        grid=(n_blocks,),
        in_specs=[
            pl.BlockSpec((BN, K), lambda i: (i, 0)),
            pl.BlockSpec((K, 1), lambda i: (0, 0)),
            pl.BlockSpec((K, 1), lambda i: (0, 0)),
        ],
        out_specs=pl.BlockSpec((BN // 128, 128), lambda i: (i, 0)),
        out_shape=jax.ShapeDtypeStruct((N_pad // 128, 128), jnp.float32),
        scratch_shapes=[pltpu.VMEM((128, 1), jnp.float32)],
    )(node_attrs, azf, crp)

    r0 = _make_gather_r0(E, TBL)(rnode4.reshape(N_pad), edge_index)

    W = 128
    rows = E // W
    RB = 5000
    ab = jnp.stack([a.astype(jnp.float32), b.astype(jnp.float32)])
    y = pl.pallas_call(
        _soft_kernel,
        grid=(rows // RB,),
        in_specs=[
            pl.BlockSpec((RB, W), lambda i: (i, 0)),
            pl.BlockSpec((RB, W), lambda i: (i, 0)),
            pl.BlockSpec(memory_space=pltpu.SMEM),
        ],
        out_specs=pl.BlockSpec((RB, W), lambda i: (i, 0)),
        out_shape=jax.ShapeDtypeStruct((rows, W), jnp.float32),
    )(x.reshape(rows, W), r0, ab)
    return y.reshape(E, 1)
